# Initial kernel scaffold; baseline (speedup 1.0000x reference)
import jax
import jax.numpy as jnp
from jax.experimental import pallas as pl


def kernel(inputs, targets):
    raise NotImplementedError("write your pallas kernel here")


# speedup vs baseline: 1.1505x; 1.1505x over previous
import functools

import jax
import jax.numpy as jnp
from jax.experimental import pallas as pl
from jax.experimental.pallas import tpu as pltpu

_LANES = 128
_SUB = 8
_EPS = 1e-6


def _partials_kernel(x_ref, y_ref, acc_ref):
    j = pl.program_id(1)

    x = x_ref[...].astype(jnp.float32)
    y = y_ref[...].astype(jnp.float32)

    t = jnp.tanh(0.5 * x)
    sig = 0.5 * t + 0.5
    sig_abs = 0.5 * jnp.abs(t) + 0.5
    bce = jnp.maximum(x, 0.0) - x * y - jnp.log(sig_abs)

    def fold(v):
        return jnp.sum(v.reshape(-1, _SUB, _LANES), axis=0)

    p_bce = fold(bce)
    p_inter = fold(sig * y)
    p_den = fold(sig + y)

    @pl.when(j == 0)
    def _init():
        acc_ref[0, 0] = p_bce
        acc_ref[0, 1] = p_inter
        acc_ref[0, 2] = p_den

    @pl.when(j > 0)
    def _accum():
        acc_ref[0, 0] += p_bce
        acc_ref[0, 1] += p_inter
        acc_ref[0, 2] += p_den


def _finalize_kernel(p_ref, out_ref, *, inv_n):
    p = p_ref[...]
    bce_sum = jnp.sum(p[:, 0])
    inter = jnp.sum(p[:, 1])
    denom = jnp.sum(p[:, 2])
    out_ref[0, 0] = (bce_sum * inv_n + 1.0) - 2.0 * inter / (denom + _EPS)


def _dice_bce(x, y, *, tile_rows=2048):
    n = int(x.size)
    rows = n // _LANES
    assert n == rows * _LANES and rows % (2 * _SUB) == 0, x.shape

    half = rows // 2
    tr = min(tile_rows, half)
    while half % tr:
        tr //= 2
    k = half // tr

    x2 = x.reshape(rows, _LANES)
    y2 = y.reshape(rows, _LANES)

    partials = pl.pallas_call(
        _partials_kernel,
        out_shape=jax.ShapeDtypeStruct((2, 3, _SUB, _LANES), jnp.float32),
        grid=(2, k),
        in_specs=[pl.BlockSpec((tr, _LANES), lambda i, j, kk=k: (i * kk + j, 0)),
                  pl.BlockSpec((tr, _LANES), lambda i, j, kk=k: (i * kk + j, 0))],
        out_specs=pl.BlockSpec((1, 3, _SUB, _LANES), lambda i, j: (i, 0, 0, 0)),
        compiler_params=pltpu.CompilerParams(
            dimension_semantics=("parallel", "arbitrary")),
    )(x2, y2)

    result = pl.pallas_call(
        functools.partial(_finalize_kernel, inv_n=1.0 / float(n)),
        out_shape=jax.ShapeDtypeStruct((1, 1), jnp.float32),
        in_specs=[pl.BlockSpec((2, 3, _SUB, _LANES), lambda: (0, 0, 0, 0))],
        out_specs=pl.BlockSpec(memory_space=pltpu.SMEM),
    )(partials)

    return result.reshape(())


def kernel(inputs, targets):
    return _dice_bce(inputs, targets)


# speedup vs baseline: 3.0580x; 2.6581x over previous
import functools

import jax
import jax.numpy as jnp
from jax.experimental import pallas as pl
from jax.experimental.pallas import tpu as pltpu

_SUB = 8
_EPS = 1e-6


def _terms(x, y):
    t = jnp.tanh(0.5 * x)
    sig = 0.5 * t + 0.5
    sig_abs = 0.5 * jnp.abs(t) + 0.5
    bce = jnp.maximum(x, 0.0) - x * y - jnp.log(sig_abs)
    return bce, sig


def _partials_kernel(xa_ref, xb_ref, ya_ref, yb_ref, acc_ref, *, width):
    j = pl.program_id(1)

    def half(x_ref, y_ref):
        x = x_ref[...].reshape(-1, _SUB, width)
        y = y_ref[...].reshape(-1, _SUB, width)
        bce, sig = _terms(x, y)
        return (jnp.sum(bce, axis=0), jnp.sum(sig * y, axis=0),
                jnp.sum(sig + y, axis=0))

    a = half(xa_ref, ya_ref)
    b = half(xb_ref, yb_ref)
    p_bce, p_inter, p_den = (a[0] + b[0], a[1] + b[1], a[2] + b[2])

    @pl.when(j == 0)
    def _init():
        acc_ref[0, 0] = p_bce
        acc_ref[0, 1] = p_inter
        acc_ref[0, 2] = p_den

    @pl.when(j > 0)
    def _accum():
        acc_ref[0, 0] += p_bce
        acc_ref[0, 1] += p_inter
        acc_ref[0, 2] += p_den


def _finalize_kernel(p_ref, out_ref, *, inv_n):
    p = p_ref[...]
    bce_sum = jnp.sum(p[:, 0])
    inter = jnp.sum(p[:, 1])
    denom = jnp.sum(p[:, 2])
    out_ref[0, 0] = (bce_sum * inv_n + 1.0) - 2.0 * inter / (denom + _EPS)


def _dice_bce(x, y, *, batch_per_step=2):
    b, c, h, w = x.shape
    assert c == 1 and w % 128 == 0 and (h // 2) % _SUB == 0 and b % 2 == 0, x.shape
    n = b * c * h * w

    cores = 2
    bb = batch_per_step
    while (b // cores) % bb:
        bb //= 2
    k = b // (cores * bb)

    h2 = h // 2
    lo = lambda i, j, kk=k: (i * kk + j, 0, 0, 0)
    hi = lambda i, j, kk=k: (i * kk + j, 0, 1, 0)

    partials = pl.pallas_call(
        functools.partial(_partials_kernel, width=w),
        out_shape=jax.ShapeDtypeStruct((cores, 3, _SUB, w), jnp.float32),
        grid=(cores, k),
        in_specs=[pl.BlockSpec((bb, 1, h2, w), lo),
                  pl.BlockSpec((bb, 1, h2, w), hi),
                  pl.BlockSpec((bb, 1, h2, w), lo),
                  pl.BlockSpec((bb, 1, h2, w), hi)],
        out_specs=pl.BlockSpec((1, 3, _SUB, w), lambda i, j: (i, 0, 0, 0)),
        compiler_params=pltpu.CompilerParams(
            dimension_semantics=("parallel", "arbitrary")),
    )(x, x, y, y)

    result = pl.pallas_call(
        functools.partial(_finalize_kernel, inv_n=1.0 / float(n)),
        out_shape=jax.ShapeDtypeStruct((1, 1), jnp.float32),
        in_specs=[pl.BlockSpec(partials.shape, lambda: (0, 0, 0, 0))],
        out_specs=pl.BlockSpec(memory_space=pltpu.SMEM),
    )(partials)

    return result.reshape(())


def kernel(inputs, targets):
    return _dice_bce(inputs, targets)


# speedup vs baseline: 3.1386x; 1.0264x over previous
import functools

import jax
import jax.numpy as jnp
from jax.experimental import pallas as pl
from jax.experimental.pallas import tpu as pltpu

_SUB = 8
_EPS = 1e-6


def _terms(x, y):
    t = jnp.tanh(0.5 * x)
    sig = 0.5 * t + 0.5
    sig_abs = 0.5 * jnp.abs(t) + 0.5
    bce = jnp.maximum(x, 0.0) - x * y - jnp.log(sig_abs)
    return bce, sig


def _partials_kernel(*refs, width):
    j = pl.program_id(1)
    acc_ref = refs[-1]
    ns = (len(refs) - 1) // 2
    xs, ys = refs[:ns], refs[ns:2 * ns]

    def half(x_ref, y_ref):
        x = x_ref[...].reshape(-1, _SUB, width)
        y = y_ref[...].reshape(-1, _SUB, width)
        bce, sig = _terms(x, y)
        return (jnp.sum(bce, axis=0), jnp.sum(sig * y, axis=0),
                jnp.sum(sig + y, axis=0))

    parts = [half(xr, yr) for xr, yr in zip(xs, ys)]
    p_bce = functools.reduce(jnp.add, [p[0] for p in parts])
    p_inter = functools.reduce(jnp.add, [p[1] for p in parts])
    p_den = functools.reduce(jnp.add, [p[2] for p in parts])

    @pl.when(j == 0)
    def _init():
        acc_ref[0, 0] = p_bce
        acc_ref[0, 1] = p_inter
        acc_ref[0, 2] = p_den

    @pl.when(j > 0)
    def _accum():
        acc_ref[0, 0] += p_bce
        acc_ref[0, 1] += p_inter
        acc_ref[0, 2] += p_den


def _finalize_kernel(p_ref, out_ref, *, inv_n):
    p = p_ref[...]
    bce_sum = jnp.sum(p[:, 0])
    inter = jnp.sum(p[:, 1])
    denom = jnp.sum(p[:, 2])
    out_ref[0, 0] = (bce_sum * inv_n + 1.0) - 2.0 * inter / (denom + _EPS)


def _dice_bce(x, y, *, batch_per_step=2):
    b, c, h, w = x.shape
    assert c == 1 and w % 128 == 0 and (h // 4) % _SUB == 0 and b % 2 == 0, x.shape
    n = b * c * h * w

    cores = 2
    bb = batch_per_step
    while (b // cores) % bb:
        bb //= 2
    k = b // (cores * bb)

    nsplit = 4
    hs = h // nsplit
    maps = [(lambda s: (lambda i, j, kk=k: (i * kk + j, 0, s, 0)))(s)
            for s in range(nsplit)]
    specs = [pl.BlockSpec((bb, 1, hs, w), m) for m in maps]

    partials = pl.pallas_call(
        functools.partial(_partials_kernel, width=w),
        out_shape=jax.ShapeDtypeStruct((cores, 3, _SUB, w), jnp.float32),
        grid=(cores, k),
        in_specs=specs + specs,
        out_specs=pl.BlockSpec((1, 3, _SUB, w), lambda i, j: (i, 0, 0, 0)),
        compiler_params=pltpu.CompilerParams(
            dimension_semantics=("parallel", "arbitrary")),
    )(*([x] * nsplit + [y] * nsplit))

    result = pl.pallas_call(
        functools.partial(_finalize_kernel, inv_n=1.0 / float(n)),
        out_shape=jax.ShapeDtypeStruct((1, 1), jnp.float32),
        in_specs=[pl.BlockSpec(partials.shape, lambda: (0, 0, 0, 0))],
        out_specs=pl.BlockSpec(memory_space=pltpu.SMEM),
    )(partials)

    return result.reshape(())


def kernel(inputs, targets):
    return _dice_bce(inputs, targets)


# speedup vs baseline: 3.2031x; 1.0205x over previous
import functools

import jax
import jax.numpy as jnp
from jax.experimental import pallas as pl
from jax.experimental.pallas import tpu as pltpu

_SUB = 8
_EPS = 1e-6


def _terms(x, y):
    t = jnp.tanh(0.5 * x)
    sig = 0.5 * t + 0.5
    sig_abs = 0.5 * jnp.abs(t) + 0.5
    bce = jnp.maximum(x, 0.0) - x * y - jnp.log(sig_abs)
    return bce, sig


def _partials_kernel(*refs, width):
    j = pl.program_id(1)
    acc_ref = refs[-1]
    ns = (len(refs) - 1) // 2
    xs, ys = refs[:ns], refs[ns:2 * ns]

    def half(x_ref, y_ref):
        x = x_ref[...].reshape(-1, _SUB, width)
        y = y_ref[...].reshape(-1, _SUB, width)
        bce, sig = _terms(x, y)
        return (jnp.sum(bce, axis=0), jnp.sum(sig * y, axis=0),
                jnp.sum(sig + y, axis=0))

    parts = [half(xr, yr) for xr, yr in zip(xs, ys)]
    p_bce = functools.reduce(jnp.add, [p[0] for p in parts])
    p_inter = functools.reduce(jnp.add, [p[1] for p in parts])
    p_den = functools.reduce(jnp.add, [p[2] for p in parts])

    @pl.when(j == 0)
    def _init():
        acc_ref[0, 0] = p_bce
        acc_ref[0, 1] = p_inter
        acc_ref[0, 2] = p_den

    @pl.when(j > 0)
    def _accum():
        acc_ref[0, 0] += p_bce
        acc_ref[0, 1] += p_inter
        acc_ref[0, 2] += p_den


def _finalize_kernel(p_ref, out_ref, *, inv_n):
    p = p_ref[...]
    bce_sum = jnp.sum(p[:, 0])
    inter = jnp.sum(p[:, 1])
    denom = jnp.sum(p[:, 2])
    out_ref[0, 0] = (bce_sum * inv_n + 1.0) - 2.0 * inter / (denom + _EPS)


def _dice_bce(x, y, *, batch_per_step=2):
    b, c, h, w = x.shape
    assert c == 1 and w % 128 == 0 and (h // 8) % _SUB == 0 and b % 2 == 0, x.shape
    n = b * c * h * w

    cores = 2
    bb = batch_per_step
    while (b // cores) % bb:
        bb //= 2
    k = b // (cores * bb)

    nsplit = 8
    hs = h // nsplit
    maps = [(lambda s: (lambda i, j, kk=k: (i * kk + j, 0, s, 0)))(s)
            for s in range(nsplit)]
    specs = [pl.BlockSpec((bb, 1, hs, w), m) for m in maps]

    partials = pl.pallas_call(
        functools.partial(_partials_kernel, width=w),
        out_shape=jax.ShapeDtypeStruct((cores, 3, _SUB, w), jnp.float32),
        grid=(cores, k),
        in_specs=specs + specs,
        out_specs=pl.BlockSpec((1, 3, _SUB, w), lambda i, j: (i, 0, 0, 0)),
        compiler_params=pltpu.CompilerParams(
            dimension_semantics=("parallel", "arbitrary")),
    )(*([x] * nsplit + [y] * nsplit))

    result = pl.pallas_call(
        functools.partial(_finalize_kernel, inv_n=1.0 / float(n)),
        out_shape=jax.ShapeDtypeStruct((1, 1), jnp.float32),
        in_specs=[pl.BlockSpec(partials.shape, lambda: (0, 0, 0, 0))],
        out_specs=pl.BlockSpec(memory_space=pltpu.SMEM),
    )(partials)

    return result.reshape(())


def kernel(inputs, targets):
    return _dice_bce(inputs, targets)


# speedup vs baseline: 3.4585x; 1.0797x over previous
import functools

import jax
import jax.numpy as jnp
from jax.experimental import pallas as pl
from jax.experimental.pallas import tpu as pltpu

_SUB = 8
_EPS = 1e-6


def _terms(x, y):
    t = jnp.tanh(0.5 * x)
    sig = 0.5 * t + 0.5
    sig_abs = 0.5 * jnp.abs(t) + 0.5
    bce = jnp.maximum(x, 0.0) - x * y - jnp.log(sig_abs)
    return bce, sig


def _partials_kernel(*refs, width):
    j = pl.program_id(1)
    acc_ref = refs[-1]
    ns = (len(refs) - 1) // 2
    xs, ys = refs[:ns], refs[ns:2 * ns]

    def half(x_ref, y_ref):
        x = x_ref[...].reshape(-1, _SUB, width)
        y = y_ref[...].reshape(-1, _SUB, width)
        bce, sig = _terms(x, y)
        return (jnp.sum(bce, axis=0), jnp.sum(sig * y, axis=0),
                jnp.sum(sig + y, axis=0))

    parts = [half(xr, yr) for xr, yr in zip(xs, ys)]
    p_bce = functools.reduce(jnp.add, [p[0] for p in parts])
    p_inter = functools.reduce(jnp.add, [p[1] for p in parts])
    p_den = functools.reduce(jnp.add, [p[2] for p in parts])

    @pl.when(j == 0)
    def _init():
        acc_ref[0, 0] = p_bce
        acc_ref[0, 1] = p_inter
        acc_ref[0, 2] = p_den

    @pl.when(j > 0)
    def _accum():
        acc_ref[0, 0] += p_bce
        acc_ref[0, 1] += p_inter
        acc_ref[0, 2] += p_den


def _finalize_kernel(p_ref, out_ref, *, inv_n):
    p = p_ref[...]
    bce_sum = jnp.sum(p[:, 0])
    inter = jnp.sum(p[:, 1])
    denom = jnp.sum(p[:, 2])
    out_ref[0, 0] = (bce_sum * inv_n + 1.0) - 2.0 * inter / (denom + _EPS)


def _dice_bce(x, y, *, batch_per_step=2):
    b, c, h, w = x.shape
    assert c == 1 and w % 128 == 0 and (h // 16) % _SUB == 0 and b % 2 == 0, x.shape
    n = b * c * h * w

    cores = 2
    bb = batch_per_step
    while (b // cores) % bb:
        bb //= 2
    k = b // (cores * bb)

    nsplit = 16
    hs = h // nsplit
    maps = [(lambda s: (lambda i, j, kk=k: (i * kk + j, 0, s, 0)))(s)
            for s in range(nsplit)]
    specs = [pl.BlockSpec((bb, 1, hs, w), m) for m in maps]

    partials = pl.pallas_call(
        functools.partial(_partials_kernel, width=w),
        out_shape=jax.ShapeDtypeStruct((cores, 3, _SUB, w), jnp.float32),
        grid=(cores, k),
        in_specs=specs + specs,
        out_specs=pl.BlockSpec((1, 3, _SUB, w), lambda i, j: (i, 0, 0, 0)),
        compiler_params=pltpu.CompilerParams(
            dimension_semantics=("parallel", "arbitrary")),
    )(*([x] * nsplit + [y] * nsplit))

    result = pl.pallas_call(
        functools.partial(_finalize_kernel, inv_n=1.0 / float(n)),
        out_shape=jax.ShapeDtypeStruct((1, 1), jnp.float32),
        in_specs=[pl.BlockSpec(partials.shape, lambda: (0, 0, 0, 0))],
        out_specs=pl.BlockSpec(memory_space=pltpu.SMEM),
    )(partials)

    return result.reshape(())


def kernel(inputs, targets):
    return _dice_bce(inputs, targets)


# speedup vs baseline: 3.4920x; 1.0097x over previous
import functools

import jax
import jax.numpy as jnp
from jax.experimental import pallas as pl
from jax.experimental.pallas import tpu as pltpu

_SUB = 8
_EPS = 1e-6


def _terms(x, y):
    t = jnp.tanh(0.5 * x)
    sig = 0.5 * t + 0.5
    sig_abs = 0.5 * jnp.abs(t) + 0.5
    bce = jnp.maximum(x, 0.0) - x * y - jnp.log(sig_abs)
    return bce, sig


def _partials_kernel(*refs, width):
    j = pl.program_id(1)
    acc_ref = refs[-1]
    ns = (len(refs) - 1) // 2
    xs, ys = refs[:ns], refs[ns:2 * ns]

    def half(x_ref, y_ref):
        x = x_ref[...].reshape(-1, _SUB, width)
        y = y_ref[...].reshape(-1, _SUB, width)
        bce, sig = _terms(x, y)
        return (jnp.sum(bce, axis=0), jnp.sum(sig * y, axis=0),
                jnp.sum(sig + y, axis=0))

    parts = [half(xr, yr) for xr, yr in zip(xs, ys)]
    p_bce = functools.reduce(jnp.add, [p[0] for p in parts])
    p_inter = functools.reduce(jnp.add, [p[1] for p in parts])
    p_den = functools.reduce(jnp.add, [p[2] for p in parts])

    @pl.when(j == 0)
    def _init():
        acc_ref[0, 0] = p_bce
        acc_ref[0, 1] = p_inter
        acc_ref[0, 2] = p_den

    @pl.when(j > 0)
    def _accum():
        acc_ref[0, 0] += p_bce
        acc_ref[0, 1] += p_inter
        acc_ref[0, 2] += p_den


def _finalize_kernel(p_ref, out_ref, *, inv_n):
    p = p_ref[...]
    bce_sum = jnp.sum(p[:, 0])
    inter = jnp.sum(p[:, 1])
    denom = jnp.sum(p[:, 2])
    out_ref[0, 0] = (bce_sum * inv_n + 1.0) - 2.0 * inter / (denom + _EPS)


def _dice_bce(x, y, *, batch_per_step=2):
    b, c, h, w = x.shape
    assert c == 1 and w % 128 == 0 and (h // 32) % _SUB == 0 and b % 2 == 0, x.shape
    n = b * c * h * w

    cores = 2
    bb = batch_per_step
    while (b // cores) % bb:
        bb //= 2
    k = b // (cores * bb)

    nsplit = 32
    hs = h // nsplit
    maps = [(lambda s: (lambda i, j, kk=k: (i * kk + j, 0, s, 0)))(s)
            for s in range(nsplit)]
    specs = [pl.BlockSpec((bb, 1, hs, w), m) for m in maps]

    partials = pl.pallas_call(
        functools.partial(_partials_kernel, width=w),
        out_shape=jax.ShapeDtypeStruct((cores, 3, _SUB, w), jnp.float32),
        grid=(cores, k),
        in_specs=specs + specs,
        out_specs=pl.BlockSpec((1, 3, _SUB, w), lambda i, j: (i, 0, 0, 0)),
        compiler_params=pltpu.CompilerParams(
            dimension_semantics=("parallel", "arbitrary")),
    )(*([x] * nsplit + [y] * nsplit))

    result = pl.pallas_call(
        functools.partial(_finalize_kernel, inv_n=1.0 / float(n)),
        out_shape=jax.ShapeDtypeStruct((1, 1), jnp.float32),
        in_specs=[pl.BlockSpec(partials.shape, lambda: (0, 0, 0, 0))],
        out_specs=pl.BlockSpec(memory_space=pltpu.SMEM),
    )(partials)

    return result.reshape(())


def kernel(inputs, targets):
    return _dice_bce(inputs, targets)
